# baseline (speedup 1.0000x reference)
import jax
import jax.numpy as jnp
from jax import lax
from jax.experimental import pallas as pl
from jax.experimental.pallas import tpu as pltpu
from jax.experimental.pallas import tpu_sc as plsc

N = 10000
NUM_MOL = 100
HIDDEN = 128
FILTERS = 128
NUM_INT = 6
NUM_G = 50
CUTOFF = 10.0
MAXNB = 32

R = 200
CW = 512
NPAD = 10624
BN = 200
NCH = 2
CHN = N // NCH
LOG2 = 0.6931471805599453


def _ssp(x):
    return jnp.maximum(x, 0.0) + jnp.log1p(jnp.exp(-jnp.abs(x))) - LOG2


def _rg_body(posr_ref, batr_ref, posc_ref, batc_ref,
             src_ref, dist_ref, c_ref, wv, wi):
    b = pl.program_id(0)
    pr = posr_ref[...]
    sqr = jnp.sum(pr * pr, axis=1, keepdims=True)
    batr = batr_ref[...]
    b0 = jnp.min(batr)
    bL = jnp.max(batr)
    colb = batc_ref[...]
    colio = lax.broadcasted_iota(jnp.int32, (1, NPAD), 1)
    lo = jnp.min(jnp.where(colb == b0, colio, NPAD))
    hi = jnp.max(jnp.where(colb == bL, colio, -1)) + 1
    s0 = (lo // 128) * 128
    nch = (hi - s0 + CW - 1) // CW
    rowio = b * R + lax.broadcasted_iota(jnp.int32, (R, 1), 0)

    wv[:, :MAXNB] = jnp.full((R, MAXNB), jnp.inf, jnp.float32)
    wi[:, :MAXNB] = -(lax.broadcasted_iota(jnp.int32, (R, MAXNB), 1) + 1)

    def chunk(c, carry):
        s = s0 + c * CW
        pc = posc_ref[:, pl.ds(s, CW)]
        sqc = jnp.sum(pc * pc, axis=0, keepdims=True)
        dot = lax.dot_general(pr, pc, (((1,), (0,)), ((), ())),
                              preferred_element_type=jnp.float32)
        d2 = jnp.maximum(sqr + sqc - 2.0 * dot, 0.0)
        gix = s + lax.broadcasted_iota(jnp.int32, (1, CW), 1)
        bc = batc_ref[:, pl.ds(s, CW)]
        valid = (bc == batr) & (gix != rowio) & (d2 <= CUTOFF * CUTOFF)
        wv[:, MAXNB:] = jnp.where(valid, d2, jnp.inf)
        wi[:, MAXNB:] = jnp.broadcast_to(gix, (R, CW))
        v = wv[...]
        gi_all = wi[...]
        ms = []
        gs = []
        for _ in range(MAXNB):
            m = jnp.min(v, axis=1, keepdims=True)
            tie = v == m
            g = jnp.min(jnp.where(tie, gi_all, jnp.int32(2**31 - 1)),
                        axis=1, keepdims=True)
            ms.append(m)
            gs.append(g)
            v = jnp.where(gi_all == g, jnp.inf, v)
        wv[:, :MAXNB] = jnp.concatenate(ms, axis=1)
        wi[:, :MAXNB] = jnp.concatenate(gs, axis=1)
        return carry

    lax.fori_loop(0, nch, chunk, 0)

    bv = wv[:, :MAXNB]
    bi = wi[:, :MAXNB]
    fin = jnp.isfinite(bv)
    dist = jnp.sqrt(jnp.where(fin, bv, 1.0))
    cmask = 0.5 * (jnp.cos(dist * (jnp.pi / CUTOFF)) + 1.0) * fin.astype(jnp.float32)
    src_ref[...] = jnp.where(fin, bi, 0)
    dist_ref[...] = dist
    c_ref[...] = cmask


def _radius_graph(pos, batch):
    posc = jnp.pad(pos, ((0, NPAD - N), (0, 0))).T
    batc = jnp.pad(batch, (0, NPAD - N),
                   constant_values=-1).reshape(1, NPAD)
    batr = batch.reshape(N, 1)
    grid = (N // R,)
    return pl.pallas_call(
        _rg_body,
        grid=grid,
        in_specs=[
            pl.BlockSpec((R, 3), lambda b: (b, 0)),
            pl.BlockSpec((R, 1), lambda b: (b, 0)),
            pl.BlockSpec((3, NPAD), lambda b: (0, 0)),
            pl.BlockSpec((1, NPAD), lambda b: (0, 0)),
        ],
        out_specs=[
            pl.BlockSpec((R, MAXNB), lambda b: (b, 0)),
            pl.BlockSpec((R, MAXNB), lambda b: (b, 0)),
            pl.BlockSpec((R, MAXNB), lambda b: (b, 0)),
        ],
        out_shape=[
            jax.ShapeDtypeStruct((N, MAXNB), jnp.int32),
            jax.ShapeDtypeStruct((N, MAXNB), jnp.float32),
            jax.ShapeDtypeStruct((N, MAXNB), jnp.float32),
        ],
        scratch_shapes=[
            pltpu.VMEM((R, MAXNB + CW), jnp.float32),
            pltpu.VMEM((R, MAXNB + CW), jnp.int32),
        ],
    )(pos, batr, posc, batc)


def _sc_gather(table, idx, window):
    n_idx = idx.shape[0]
    vdim = table.shape[1]
    mesh = plsc.VectorSubcoreMesh(core_axis_name="core",
                                  subcore_axis_name="subcore")
    idx2 = idx.reshape(1, n_idx)

    @pl.kernel(out_type=jax.ShapeDtypeStruct((n_idx, vdim), table.dtype),
               mesh=mesh,
               scratch_types=[pltpu.VMEM_SHARED(table.shape, table.dtype)])
    def k(x_hbm, i_hbm, o_hbm, shared):
        sid = lax.axis_index("subcore")

        @pl.when(sid == 0)
        def _():
            pltpu.sync_copy(x_hbm, shared)

        plsc.subcore_barrier()

        def body(i_vmem, o_vmem):
            pltpu.sync_copy(shared.at[i_vmem.at[0]], o_vmem)

        pltpu.emit_pipeline(
            body,
            grid=(n_idx // window,),
            in_specs=[pl.BlockSpec((1, window), index_map=lambda i: (0, i))],
            out_specs=[pl.BlockSpec((window, vdim), index_map=lambda i: (i, 0))],
            core_axis_name=("core", "subcore"),
            dimension_semantics=(pltpu.PARALLEL,),
        )(i_hbm, o_hbm)

    return k(table, idx2)


def _filter_agg(xsg_ref, dist_ref, c_ref, w1d_ref, b1d_ref, w2d_ref, b2d_ref):
    offs2 = jnp.tile(
        lax.broadcasted_iota(jnp.int32, (1, NUM_G), 1).astype(jnp.float32)
        * (CUTOFF / (NUM_G - 1)),
        (1, 2))
    coeff = -0.5 / (CUTOFF / (NUM_G - 1)) ** 2
    d = dist_ref[...]
    c = c_ref[...]
    agg = jnp.zeros((BN, FILTERS), jnp.float32)
    for k in range(0, MAXNB, 2):
        d2c = jnp.concatenate(
            [jnp.broadcast_to(d[:, k:k + 1], (BN, NUM_G)),
             jnp.broadcast_to(d[:, k + 1:k + 2], (BN, NUM_G))], axis=1)
        ea2 = jnp.exp(coeff * (d2c - offs2) ** 2)
        t2 = _ssp(jnp.dot(ea2.astype(jnp.bfloat16), w1d_ref[...],
                          preferred_element_type=jnp.float32) + b1d_ref[...])
        wf2 = jnp.dot(t2.astype(jnp.bfloat16), w2d_ref[...],
                      preferred_element_type=jnp.float32) + b2d_ref[...]
        agg = agg + xsg_ref[k] * (wf2[:, :FILTERS] * c[:, k:k + 1])
        agg = agg + xsg_ref[k + 1] * (wf2[:, FILTERS:] * c[:, k + 1:k + 2])
    return agg


def _inter_body(xsg_ref, dist_ref, c_ref, h_ref,
                w1d_ref, b1d_ref, w2d_ref, b2d_ref,
                l2w_ref, l2b_ref, lw_ref, lb_ref, nxt_ref,
                hn_ref, xs_ref):
    agg = _filter_agg(xsg_ref, dist_ref, c_ref, w1d_ref, b1d_ref,
                      w2d_ref, b2d_ref)
    v = _ssp(jnp.dot(agg, l2w_ref[...],
                     preferred_element_type=jnp.float32) + l2b_ref[...])
    v = jnp.dot(v, lw_ref[...], preferred_element_type=jnp.float32) + lb_ref[...]
    hn = h_ref[...] + v
    hn_ref[...] = hn
    xs_ref[...] = jnp.dot(hn, nxt_ref[...], preferred_element_type=jnp.float32)


def _final_body(xsg_ref, dist_ref, c_ref, h_ref,
                w1d_ref, b1d_ref, w2d_ref, b2d_ref,
                l2w_ref, l2b_ref, lw_ref, lb_ref,
                o1w_ref, o1b_ref, o2w_ref, o2b_ref, bat_ref,
                out_ref):
    b = pl.program_id(0)
    agg = _filter_agg(xsg_ref, dist_ref, c_ref, w1d_ref, b1d_ref,
                      w2d_ref, b2d_ref)
    v = _ssp(jnp.dot(agg, l2w_ref[...],
                     preferred_element_type=jnp.float32) + l2b_ref[...])
    v = jnp.dot(v, lw_ref[...], preferred_element_type=jnp.float32) + lb_ref[...]
    hn = h_ref[...] + v
    y = _ssp(jnp.dot(hn, o1w_ref[...],
                     preferred_element_type=jnp.float32) + o1b_ref[...])
    y = jnp.dot(y, o2w_ref[...], preferred_element_type=jnp.float32) + o2b_ref[...]
    g = (bat_ref[...] == lax.broadcasted_iota(jnp.int32, (1, NUM_MOL), 1))
    part = lax.dot_general(g.astype(jnp.float32), y, (((0,), (0,)), ((), ())),
                           precision=lax.Precision.HIGHEST,
                           preferred_element_type=jnp.float32)

    @pl.when(b == 0)
    def _():
        out_ref[...] = jnp.zeros((NUM_MOL, 1), jnp.float32)

    out_ref[...] += part


def _weight_specs():
    return [
        pl.BlockSpec((2 * NUM_G, 2 * FILTERS), lambda b: (0, 0)),

        pl.BlockSpec((1, 2 * FILTERS), lambda b: (0, 0)),
        pl.BlockSpec((2 * FILTERS, 2 * FILTERS), lambda b: (0, 0)),
        pl.BlockSpec((1, 2 * FILTERS), lambda b: (0, 0)),
        pl.BlockSpec((FILTERS, HIDDEN), lambda b: (0, 0)),
        pl.BlockSpec((1, HIDDEN), lambda b: (0, 0)),
        pl.BlockSpec((HIDDEN, HIDDEN), lambda b: (0, 0)),
        pl.BlockSpec((1, HIDDEN), lambda b: (0, 0)),
    ]


_EDGE_SPECS = [
    pl.BlockSpec((MAXNB, BN, FILTERS), lambda b: (0, b, 0)),
    pl.BlockSpec((BN, MAXNB), lambda b: (b, 0)),
    pl.BlockSpec((BN, MAXNB), lambda b: (b, 0)),
    pl.BlockSpec((BN, HIDDEN), lambda b: (b, 0)),
]


def _interaction(xsg3, dist, cmask, h, wd, nxt_w):
    m = xsg3.shape[1]
    grid = (m // BN,)
    return pl.pallas_call(
        _inter_body,
        grid=grid,
        in_specs=_EDGE_SPECS + _weight_specs() + [
            pl.BlockSpec((HIDDEN, FILTERS), lambda b: (0, 0)),
        ],
        out_specs=[
            pl.BlockSpec((BN, HIDDEN), lambda b: (b, 0)),
            pl.BlockSpec((BN, FILTERS), lambda b: (b, 0)),
        ],
        out_shape=[
            jax.ShapeDtypeStruct((m, HIDDEN), jnp.float32),
            jax.ShapeDtypeStruct((m, FILTERS), jnp.float32),
        ],
    )(xsg3, dist, cmask, h, *wd, nxt_w)


def _final(xsg3, dist, cmask, h, wd, o1w, o1b, o2w, o2b, batch):
    m = xsg3.shape[1]
    grid = (m // BN,)
    return pl.pallas_call(
        _final_body,
        grid=grid,
        in_specs=_EDGE_SPECS + _weight_specs() + [
            pl.BlockSpec((HIDDEN, HIDDEN // 2), lambda b: (0, 0)),
            pl.BlockSpec((1, HIDDEN // 2), lambda b: (0, 0)),
            pl.BlockSpec((HIDDEN // 2, 1), lambda b: (0, 0)),
            pl.BlockSpec((1, 1), lambda b: (0, 0)),
            pl.BlockSpec((BN, 1), lambda b: (b, 0)),
        ],
        out_specs=[pl.BlockSpec((NUM_MOL, 1), lambda b: (0, 0))],
        out_shape=[jax.ShapeDtypeStruct((NUM_MOL, 1), jnp.float32)],
    )(xsg3, dist, cmask, h, *wd, o1w, o1b.reshape(1, -1), o2w,
      o2b.reshape(1, 1), batch.reshape(m, 1))[0]


def _xs0_body(h_ref, w_ref, xs_ref):
    xs_ref[...] = jnp.dot(h_ref[...], w_ref[...],
                          preferred_element_type=jnp.float32)


def _xs0(h0, w):
    return pl.pallas_call(
        _xs0_body,
        grid=(N // 1000,),
        in_specs=[pl.BlockSpec((1000, HIDDEN), lambda b: (b, 0)),
                  pl.BlockSpec((HIDDEN, FILTERS), lambda b: (0, 0))],
        out_specs=pl.BlockSpec((1000, FILTERS), lambda b: (b, 0)),
        out_shape=jax.ShapeDtypeStruct((N, FILTERS), jnp.float32),
    )(h0, w)


def _blkdiag2(w):
    z = jnp.zeros_like(w)
    return jnp.concatenate(
        [jnp.concatenate([w, z], axis=1), jnp.concatenate([z, w], axis=1)],
        axis=0)


def kernel(z, pos, batch, emb, mlp_w1, mlp_b1, mlp_w2, mlp_b2,
           conv_lin1_w, conv_lin2_w, conv_lin2_b, lin_w, lin_b,
           out1_w, out1_b, out2_w, out2_b):
    src, dist, cmask = _radius_graph(pos, batch)
    src_t = src.T

    z_pad = jnp.pad(z.astype(jnp.int32), (0, 10240 - N))
    h = _sc_gather(emb, z_pad, 128)[:N]
    xs = _xs0(h, conv_lin1_w[0])

    for i in range(NUM_INT):
        wd = (
            _blkdiag2(mlp_w1[i]).astype(jnp.bfloat16),
            jnp.tile(mlp_b1[i].reshape(1, -1), (1, 2)),
            _blkdiag2(mlp_w2[i]).astype(jnp.bfloat16),
            jnp.tile(mlp_b2[i].reshape(1, -1), (1, 2)),
            conv_lin2_w[i],
            conv_lin2_b[i].reshape(1, -1),
            lin_w[i],
            lin_b[i].reshape(1, -1),
        )
        h_parts, xs_parts, out_parts = [], [], []
        for c in range(NCH):
            sl = slice(c * CHN, (c + 1) * CHN)
            idx_c = src_t[:, sl].reshape(MAXNB * CHN)
            xsg = _sc_gather(xs, idx_c, 128).reshape(MAXNB, CHN, FILTERS)
            if i < NUM_INT - 1:
                hc, xsc = _interaction(xsg, dist[sl], cmask[sl], h[sl], wd,
                                       conv_lin1_w[i + 1])
                h_parts.append(hc)
                xs_parts.append(xsc)
            else:
                out_parts.append(_final(xsg, dist[sl], cmask[sl], h[sl], wd,
                                        out1_w, out1_b, out2_w, out2_b,
                                        batch[sl]))
        if i < NUM_INT - 1:
            h = jnp.concatenate(h_parts, axis=0)
            xs = jnp.concatenate(xs_parts, axis=0)
        else:
            out = sum(out_parts)
    return out.reshape(-1)


# speedup vs baseline: 1.0122x; 1.0122x over previous
import jax
import jax.numpy as jnp
from jax import lax
from jax.experimental import pallas as pl
from jax.experimental.pallas import tpu as pltpu
from jax.experimental.pallas import tpu_sc as plsc

N = 10000
NUM_MOL = 100
HIDDEN = 128
FILTERS = 128
NUM_INT = 6
NUM_G = 50
CUTOFF = 10.0
MAXNB = 32

R = 200
CW = 512
NPAD = 10624
BN = 200
NCH = 2
CHN = N // NCH
LOG2 = 0.6931471805599453


def _ssp(x):
    return jnp.maximum(x, 0.0) + jnp.log1p(jnp.exp(-jnp.abs(x))) - LOG2


def _rg_body(posr_ref, batr_ref, posc_ref, batc_ref,
             src_ref, dist_ref, c_ref, wv, wi):
    b = pl.program_id(0)
    pr = posr_ref[...]
    sqr = jnp.sum(pr * pr, axis=1, keepdims=True)
    batr = batr_ref[...]
    b0 = jnp.min(batr)
    bL = jnp.max(batr)
    colb = batc_ref[...]
    colio = lax.broadcasted_iota(jnp.int32, (1, NPAD), 1)
    lo = jnp.min(jnp.where(colb == b0, colio, NPAD))
    hi = jnp.max(jnp.where(colb == bL, colio, -1)) + 1
    s0 = (lo // 128) * 128
    nch = (hi - s0 + CW - 1) // CW
    rowio = b * R + lax.broadcasted_iota(jnp.int32, (R, 1), 0)

    wv[:, :MAXNB] = jnp.full((R, MAXNB), jnp.inf, jnp.float32)
    wi[:, :MAXNB] = -(lax.broadcasted_iota(jnp.int32, (R, MAXNB), 1) + 1)

    def chunk(c, carry):
        s = s0 + c * CW
        pc = posc_ref[:, pl.ds(s, CW)]
        sqc = jnp.sum(pc * pc, axis=0, keepdims=True)
        dot = lax.dot_general(pr, pc, (((1,), (0,)), ((), ())),
                              preferred_element_type=jnp.float32)
        d2 = jnp.maximum(sqr + sqc - 2.0 * dot, 0.0)
        gix = s + lax.broadcasted_iota(jnp.int32, (1, CW), 1)
        bc = batc_ref[:, pl.ds(s, CW)]
        valid = (bc == batr) & (gix != rowio) & (d2 <= CUTOFF * CUTOFF)
        wv[:, MAXNB:] = jnp.where(valid, d2, jnp.inf)
        wi[:, MAXNB:] = jnp.broadcast_to(gix, (R, CW))
        v = wv[...]
        gi_all = wi[...]
        ms = []
        gs = []
        for _ in range(MAXNB):
            m = jnp.min(v, axis=1, keepdims=True)
            tie = v == m
            g = jnp.min(jnp.where(tie, gi_all, jnp.int32(2**31 - 1)),
                        axis=1, keepdims=True)
            ms.append(m)
            gs.append(g)
            v = jnp.where(gi_all == g, jnp.inf, v)
        wv[:, :MAXNB] = jnp.concatenate(ms, axis=1)
        wi[:, :MAXNB] = jnp.concatenate(gs, axis=1)
        return carry

    lax.fori_loop(0, nch, chunk, 0)

    bv = wv[:, :MAXNB]
    bi = wi[:, :MAXNB]
    fin = jnp.isfinite(bv)
    dist = jnp.sqrt(jnp.where(fin, bv, 1.0))
    cmask = 0.5 * (jnp.cos(dist * (jnp.pi / CUTOFF)) + 1.0) * fin.astype(jnp.float32)
    src_ref[...] = jnp.where(fin, bi, 0)
    dist_ref[...] = dist
    c_ref[...] = cmask


def _radius_graph(pos, batch):
    posc = jnp.pad(pos, ((0, NPAD - N), (0, 0))).T
    batc = jnp.pad(batch, (0, NPAD - N),
                   constant_values=-1).reshape(1, NPAD)
    batr = batch.reshape(N, 1)
    grid = (N // R,)
    return pl.pallas_call(
        _rg_body,
        grid=grid,
        in_specs=[
            pl.BlockSpec((R, 3), lambda b: (b, 0)),
            pl.BlockSpec((R, 1), lambda b: (b, 0)),
            pl.BlockSpec((3, NPAD), lambda b: (0, 0)),
            pl.BlockSpec((1, NPAD), lambda b: (0, 0)),
        ],
        out_specs=[
            pl.BlockSpec((R, MAXNB), lambda b: (b, 0)),
            pl.BlockSpec((R, MAXNB), lambda b: (b, 0)),
            pl.BlockSpec((R, MAXNB), lambda b: (b, 0)),
        ],
        out_shape=[
            jax.ShapeDtypeStruct((N, MAXNB), jnp.int32),
            jax.ShapeDtypeStruct((N, MAXNB), jnp.float32),
            jax.ShapeDtypeStruct((N, MAXNB), jnp.float32),
        ],
        scratch_shapes=[
            pltpu.VMEM((R, MAXNB + CW), jnp.float32),
            pltpu.VMEM((R, MAXNB + CW), jnp.int32),
        ],
    )(pos, batr, posc, batc)


def _sc_gather(table, idx, window):
    n_idx = idx.shape[0]
    vdim = table.shape[1]
    mesh = plsc.VectorSubcoreMesh(core_axis_name="core",
                                  subcore_axis_name="subcore")
    idx2 = idx.reshape(1, n_idx)

    @pl.kernel(out_type=jax.ShapeDtypeStruct((n_idx, vdim), table.dtype),
               mesh=mesh,
               scratch_types=[pltpu.VMEM_SHARED(table.shape, table.dtype)])
    def k(x_hbm, i_hbm, o_hbm, shared):
        sid = lax.axis_index("subcore")

        @pl.when(sid == 0)
        def _():
            pltpu.sync_copy(x_hbm, shared)

        plsc.subcore_barrier()

        def body(i_vmem, o_vmem):
            pltpu.sync_copy(shared.at[i_vmem.at[0]], o_vmem)

        pltpu.emit_pipeline(
            body,
            grid=(n_idx // window,),
            in_specs=[pl.BlockSpec((1, window), index_map=lambda i: (0, i))],
            out_specs=[pl.BlockSpec((window, vdim), index_map=lambda i: (i, 0))],
            core_axis_name=("core", "subcore"),
            dimension_semantics=(pltpu.PARALLEL,),
        )(i_hbm, o_hbm)

    return k(table, idx2)


def _filter_agg(xsg_ref, dist_ref, c_ref, w1d_ref, b1d_ref, w2d_ref, b2d_ref):
    offs2 = jnp.tile(
        lax.broadcasted_iota(jnp.int32, (1, NUM_G), 1).astype(jnp.float32)
        * (CUTOFF / (NUM_G - 1)),
        (1, 2))
    coeff = -0.5 / (CUTOFF / (NUM_G - 1)) ** 2
    d = dist_ref[...]
    c = c_ref[...]
    agg = jnp.zeros((BN, FILTERS), jnp.float32)
    for k in range(0, MAXNB, 2):
        d2c = jnp.concatenate(
            [jnp.broadcast_to(d[:, k:k + 1], (BN, NUM_G)),
             jnp.broadcast_to(d[:, k + 1:k + 2], (BN, NUM_G))], axis=1)
        ea2 = jnp.exp(coeff * (d2c - offs2) ** 2)
        t2 = _ssp(jnp.dot(ea2, w1d_ref[...],
                          preferred_element_type=jnp.float32) + b1d_ref[...])
        wf2 = jnp.dot(t2, w2d_ref[...],
                      preferred_element_type=jnp.float32) + b2d_ref[...]
        agg = agg + xsg_ref[k] * (wf2[:, :FILTERS] * c[:, k:k + 1])
        agg = agg + xsg_ref[k + 1] * (wf2[:, FILTERS:] * c[:, k + 1:k + 2])
    return agg


def _inter_body(xsg_ref, dist_ref, c_ref, h_ref,
                w1d_ref, b1d_ref, w2d_ref, b2d_ref,
                l2w_ref, l2b_ref, lw_ref, lb_ref, nxt_ref,
                hn_ref, xs_ref):
    agg = _filter_agg(xsg_ref, dist_ref, c_ref, w1d_ref, b1d_ref,
                      w2d_ref, b2d_ref)
    v = _ssp(jnp.dot(agg, l2w_ref[...],
                     preferred_element_type=jnp.float32) + l2b_ref[...])
    v = jnp.dot(v, lw_ref[...], preferred_element_type=jnp.float32) + lb_ref[...]
    hn = h_ref[...] + v
    hn_ref[...] = hn
    xs_ref[...] = jnp.dot(hn, nxt_ref[...], preferred_element_type=jnp.float32)


def _final_body(xsg_ref, dist_ref, c_ref, h_ref,
                w1d_ref, b1d_ref, w2d_ref, b2d_ref,
                l2w_ref, l2b_ref, lw_ref, lb_ref,
                o1w_ref, o1b_ref, o2w_ref, o2b_ref, bat_ref,
                out_ref):
    b = pl.program_id(0)
    agg = _filter_agg(xsg_ref, dist_ref, c_ref, w1d_ref, b1d_ref,
                      w2d_ref, b2d_ref)
    v = _ssp(jnp.dot(agg, l2w_ref[...],
                     preferred_element_type=jnp.float32) + l2b_ref[...])
    v = jnp.dot(v, lw_ref[...], preferred_element_type=jnp.float32) + lb_ref[...]
    hn = h_ref[...] + v
    y = _ssp(jnp.dot(hn, o1w_ref[...],
                     preferred_element_type=jnp.float32) + o1b_ref[...])
    y = jnp.dot(y, o2w_ref[...], preferred_element_type=jnp.float32) + o2b_ref[...]
    g = (bat_ref[...] == lax.broadcasted_iota(jnp.int32, (1, NUM_MOL), 1))
    part = lax.dot_general(g.astype(jnp.float32), y, (((0,), (0,)), ((), ())),
                           precision=lax.Precision.HIGHEST,
                           preferred_element_type=jnp.float32)

    @pl.when(b == 0)
    def _():
        out_ref[...] = jnp.zeros((NUM_MOL, 1), jnp.float32)

    out_ref[...] += part


def _weight_specs():
    return [
        pl.BlockSpec((2 * NUM_G, 2 * FILTERS), lambda b: (0, 0)),
        pl.BlockSpec((1, 2 * FILTERS), lambda b: (0, 0)),
        pl.BlockSpec((2 * FILTERS, 2 * FILTERS), lambda b: (0, 0)),
        pl.BlockSpec((1, 2 * FILTERS), lambda b: (0, 0)),
        pl.BlockSpec((FILTERS, HIDDEN), lambda b: (0, 0)),
        pl.BlockSpec((1, HIDDEN), lambda b: (0, 0)),
        pl.BlockSpec((HIDDEN, HIDDEN), lambda b: (0, 0)),
        pl.BlockSpec((1, HIDDEN), lambda b: (0, 0)),
    ]


_EDGE_SPECS = [
    pl.BlockSpec((MAXNB, BN, FILTERS), lambda b: (0, b, 0)),
    pl.BlockSpec((BN, MAXNB), lambda b: (b, 0)),
    pl.BlockSpec((BN, MAXNB), lambda b: (b, 0)),
    pl.BlockSpec((BN, HIDDEN), lambda b: (b, 0)),
]


def _interaction(xsg3, dist, cmask, h, wd, nxt_w):
    m = xsg3.shape[1]
    grid = (m // BN,)
    return pl.pallas_call(
        _inter_body,
        grid=grid,
        in_specs=_EDGE_SPECS + _weight_specs() + [
            pl.BlockSpec((HIDDEN, FILTERS), lambda b: (0, 0)),
        ],
        out_specs=[
            pl.BlockSpec((BN, HIDDEN), lambda b: (b, 0)),
            pl.BlockSpec((BN, FILTERS), lambda b: (b, 0)),
        ],
        out_shape=[
            jax.ShapeDtypeStruct((m, HIDDEN), jnp.float32),
            jax.ShapeDtypeStruct((m, FILTERS), jnp.float32),
        ],
    )(xsg3, dist, cmask, h, *wd, nxt_w)


def _final(xsg3, dist, cmask, h, wd, o1w, o1b, o2w, o2b, batch):
    m = xsg3.shape[1]
    grid = (m // BN,)
    return pl.pallas_call(
        _final_body,
        grid=grid,
        in_specs=_EDGE_SPECS + _weight_specs() + [
            pl.BlockSpec((HIDDEN, HIDDEN // 2), lambda b: (0, 0)),
            pl.BlockSpec((1, HIDDEN // 2), lambda b: (0, 0)),
            pl.BlockSpec((HIDDEN // 2, 1), lambda b: (0, 0)),
            pl.BlockSpec((1, 1), lambda b: (0, 0)),
            pl.BlockSpec((BN, 1), lambda b: (b, 0)),
        ],
        out_specs=[pl.BlockSpec((NUM_MOL, 1), lambda b: (0, 0))],
        out_shape=[jax.ShapeDtypeStruct((NUM_MOL, 1), jnp.float32)],
    )(xsg3, dist, cmask, h, *wd, o1w, o1b.reshape(1, -1), o2w,
      o2b.reshape(1, 1), batch.reshape(m, 1))[0]


def _xs0_body(h_ref, w_ref, xs_ref):
    xs_ref[...] = jnp.dot(h_ref[...], w_ref[...],
                          preferred_element_type=jnp.float32)


def _xs0(h0, w):
    return pl.pallas_call(
        _xs0_body,
        grid=(N // 1000,),
        in_specs=[pl.BlockSpec((1000, HIDDEN), lambda b: (b, 0)),
                  pl.BlockSpec((HIDDEN, FILTERS), lambda b: (0, 0))],
        out_specs=pl.BlockSpec((1000, FILTERS), lambda b: (b, 0)),
        out_shape=jax.ShapeDtypeStruct((N, FILTERS), jnp.float32),
    )(h0, w)


def _blkdiag2(w):
    z = jnp.zeros_like(w)
    return jnp.concatenate(
        [jnp.concatenate([w, z], axis=1), jnp.concatenate([z, w], axis=1)],
        axis=0)


def kernel(z, pos, batch, emb, mlp_w1, mlp_b1, mlp_w2, mlp_b2,
           conv_lin1_w, conv_lin2_w, conv_lin2_b, lin_w, lin_b,
           out1_w, out1_b, out2_w, out2_b):
    src, dist, cmask = _radius_graph(pos, batch)
    src_t = src.T

    z_pad = jnp.pad(z.astype(jnp.int32), (0, 10240 - N))
    h = _sc_gather(emb, z_pad, 128)[:N]
    xs = _xs0(h, conv_lin1_w[0])

    for i in range(NUM_INT):
        wd = (
            _blkdiag2(mlp_w1[i]),
            jnp.tile(mlp_b1[i].reshape(1, -1), (1, 2)),
            _blkdiag2(mlp_w2[i]),
            jnp.tile(mlp_b2[i].reshape(1, -1), (1, 2)),
            conv_lin2_w[i],
            conv_lin2_b[i].reshape(1, -1),
            lin_w[i],
            lin_b[i].reshape(1, -1),
        )
        h_parts, xs_parts, out_parts = [], [], []
        for c in range(NCH):
            sl = slice(c * CHN, (c + 1) * CHN)
            idx_c = src_t[:, sl].reshape(MAXNB * CHN)
            xsg = _sc_gather(xs, idx_c, 128).reshape(MAXNB, CHN, FILTERS)
            if i < NUM_INT - 1:
                hc, xsc = _interaction(xsg, dist[sl], cmask[sl], h[sl], wd,
                                       conv_lin1_w[i + 1])
                h_parts.append(hc)
                xs_parts.append(xsc)
            else:
                out_parts.append(_final(xsg, dist[sl], cmask[sl], h[sl], wd,
                                        out1_w, out1_b, out2_w, out2_b,
                                        batch[sl]))
        if i < NUM_INT - 1:
            h = jnp.concatenate(h_parts, axis=0)
            xs = jnp.concatenate(xs_parts, axis=0)
        else:
            out = sum(out_parts)
    return out.reshape(-1)
